# Initial kernel scaffold; baseline (speedup 1.0000x reference)
import jax
import jax.numpy as jnp
from jax.experimental import pallas as pl


def kernel(node_feats, edge_feats, edge_index, W_i, msg_W1, msg_b1, msg_W2, msg_b2, attn_W1, attn_b1, attn_W2, attn_b2, gru_Wih, gru_bih, gru_Whh, gru_bhh):
    raise NotImplementedError("write your pallas kernel here")


# speedup vs baseline: 42.2021x; 42.2021x over previous
import jax
import jax.numpy as jnp
from jax import lax
from jax.experimental import pallas as pl
from jax.experimental.pallas import tpu as pltpu
from jax.experimental.pallas import tpu_sc as plsc

N = 10000
NP = 10240
E = 160000
H = 8
F = 2 * H * H
NC = 2
NS = 16
NW = NC * NS
EPW = E // NW
CH = 40
NIT = EPW // CH
NPT = NP // NS

_f32 = jnp.float32
_MESH = dict(core_axis_name="c", subcore_axis_name="s")


def _sc_scatter(payload, dstidx, f):
    def body(p_hbm, idx_hbm, z_hbm, out_hbm, idx_v, rows_v, acc_sh):
        c = lax.axis_index("c")
        s = lax.axis_index("s")
        wid = c * NS + s
        base0 = wid * EPW
        nsl = pl.ds(s * NPT, NPT)
        pltpu.sync_copy(z_hbm.at[nsl], acc_sh.at[nsl])
        plsc.subcore_barrier()

        def step(i, carry):
            b = base0 + i * CH
            pltpu.sync_copy(idx_hbm.at[pl.ds(b, CH)], idx_v)
            pltpu.sync_copy(p_hbm.at[pl.ds(b, CH)], rows_v)
            pltpu.sync_copy(rows_v, acc_sh.at[idx_v], add=True)
            return carry

        lax.fori_loop(0, NIT, step, 0)
        plsc.subcore_barrier()
        pltpu.sync_copy(acc_sh.at[nsl], out_hbm.at[pl.ds(c * NP + s * NPT, NPT)])

    zeros = jnp.zeros((NP, f), _f32)
    return pl.kernel(
        body,
        out_type=jax.ShapeDtypeStruct((2 * NP, f), _f32),
        mesh=plsc.VectorSubcoreMesh(**_MESH),
        scratch_types=[
            pltpu.VMEM((CH,), jnp.int32),
            pltpu.VMEM((CH, f), _f32),
            pltpu.VMEM_SHARED((NP, f), _f32),
        ],
    )(payload, dstidx, zeros)


def _sc_gather(table, srcidx, f):
    def body(t_hbm, idx_hbm, out_hbm, idx_v, rows_v, tbl_sh, sem):
        c = lax.axis_index("c")
        s = lax.axis_index("s")
        wid = c * NS + s
        base0 = wid * EPW
        nsl = pl.ds(s * NPT, NPT)
        pltpu.sync_copy(t_hbm.at[nsl], tbl_sh.at[nsl])
        plsc.subcore_barrier()

        def step(i, carry):
            b = base0 + i * CH
            pltpu.sync_copy(idx_hbm.at[pl.ds(b, CH)], idx_v)
            pltpu.async_copy(tbl_sh.at[idx_v], rows_v, sem).wait()
            pltpu.sync_copy(rows_v, out_hbm.at[pl.ds(b, CH)])
            return carry

        lax.fori_loop(0, NIT, step, 0)

    return pl.kernel(
        body,
        out_type=jax.ShapeDtypeStruct((E, f), _f32),
        mesh=plsc.VectorSubcoreMesh(**_MESH),
        scratch_types=[
            pltpu.VMEM((CH,), jnp.int32),
            pltpu.VMEM((CH, f), _f32),
            pltpu.VMEM_SHARED((NP, f), _f32),
            pltpu.SemaphoreType.DMA,
        ],
    )(table, srcidx)


BE = 8000
GE = E // BE


def _full(shape):
    nd = len(shape)
    return pl.BlockSpec(shape, lambda i: (0,) * nd)


def _blk(shape):
    return pl.BlockSpec(shape, lambda i: (i,) + (0,) * (len(shape) - 1))


def _node_proj(node_feats, wab):
    def body(nf_ref, w_ref, out_ref):
        ab = jnp.dot(nf_ref[...], w_ref[...], preferred_element_type=_f32)
        out_ref[...] = jnp.concatenate(
            [ab, jnp.zeros((ab.shape[0], F - 2 * H), _f32)], axis=1)

    return pl.pallas_call(
        body,
        grid=(10,),
        in_specs=[_blk((N // 10, 128)), _full((128, 2 * H))],
        out_specs=_blk((N // 10, F)),
        out_shape=jax.ShapeDtypeStruct((NP, F), _f32),
    )(node_feats, wab)


def _init_ef(g_s, g_d, edge_feats, wc):
    def body(s_ref, d_ref, ef_ref, w_ref, out_ref):
        x = (s_ref[...][:, :H] + d_ref[...][:, H:2 * H]
             + jnp.dot(ef_ref[...], w_ref[...], preferred_element_type=_f32))
        x = jnp.maximum(x, 0.0)
        out_ref[...] = jnp.concatenate([x, jnp.zeros_like(x)], axis=1)

    return pl.pallas_call(
        body,
        grid=(GE,),
        in_specs=[_blk((BE, F)), _blk((BE, F)), _blk((BE, 16)),
                  _full((16, H))],
        out_specs=_blk((BE, 2 * H)),
        out_shape=jax.ShapeDtypeStruct((E, 2 * H), _f32),
    )(g_s, g_d, edge_feats, wc)


def _edge_weights(ef, w1m, b1m, w2m, b2m, w1a, b1a, w2a, b2a, rm):
    t_m = jnp.maximum(jnp.dot(ef, w1m, preferred_element_type=_f32) + b1m, 0.0)
    w_m = jnp.dot(t_m, w2m, preferred_element_type=_f32) + b2m
    t_a = jnp.maximum(jnp.dot(ef, w1a, preferred_element_type=_f32) + b1a, 0.0)
    w_a = jnp.dot(t_a, w2a, preferred_element_type=_f32) + b2a
    ef_r = jnp.dot(ef, rm, preferred_element_type=_f32)
    return w_m, w_a, ef_r


def _pass1(ef16, mw):
    def body(ef_ref, w1m, b1m, w2m, b2m, w1a, b1a, w2a, b2a, rm, out_ref):
        ef = ef_ref[...][:, :H]
        w_m, w_a, ef_r = _edge_weights(ef, w1m[...], b1m[...], w2m[...],
                                       b2m[...], w1a[...], b1a[...],
                                       w2a[...], b2a[...], rm[...])
        exp_e2 = jnp.exp(w_a * ef_r)
        h1 = exp_e2 * (w_m * ef_r)
        out_ref[...] = jnp.concatenate([exp_e2, h1], axis=1)

    wspecs = [_full(w.shape) for w in mw]
    return pl.pallas_call(
        body,
        grid=(GE,),
        in_specs=[_blk((BE, 2 * H))] + wspecs,
        out_specs=_blk((BE, F)),
        out_shape=jax.ShapeDtypeStruct((E, F), _f32),
    )(ef16, *mw)


def _combine_partials(partials, f):
    def body(a_ref, b_ref, out_ref):
        out_ref[...] = a_ref[...] + b_ref[...]

    return pl.pallas_call(
        body,
        grid=(10,),
        in_specs=[
            pl.BlockSpec((NP // 10, f), lambda i: (i, 0)),
            pl.BlockSpec((NP // 10, f), lambda i: (i + 10, 0)),
        ],
        out_specs=pl.BlockSpec((NP // 10, f), lambda i: (i, 0)),
        out_shape=jax.ShapeDtypeStruct((NP, f), _f32),
    )(partials, partials)


def _pass2_gru(g, ef16, ief16, mw, rt, gw, out_f=2 * H):
    def body(g_ref, ef_ref, ief_ref, w1m, b1m, w2m, b2m, w1a, b1a, w2a, b2a,
             rm, rt_ref, wir, wiz, win, whr, whz, whn, bir, biz, bin_,
             bhr, bhz, bhn, out_ref):
        ef = ef_ref[...][:, :H]
        ief = ief_ref[...][:, :H]
        w_m, w_a, ef_r = _edge_weights(ef, w1m[...], b1m[...], w2m[...],
                                       b2m[...], w1a[...], b1a[...],
                                       w2a[...], b2a[...], rm[...])
        ief_r = jnp.dot(ief, rm[...], preferred_element_type=_f32)
        exp_e2 = jnp.exp(w_a * ef_r)
        h1 = exp_e2 * (w_m * ef_r)
        exp_ie2 = jnp.exp(w_a * ief_r)
        ih1 = exp_ie2 * (w_m * ief_r)
        gathered = g_ref[...]
        sg = gathered[:, :H * H]
        mg = gathered[:, H * H:]
        h2 = (mg - h1 + ih1) / (sg - exp_e2 + exp_ie2)
        conv = jnp.maximum(jnp.dot(h2, rt_ref[...],
                                   preferred_element_type=_f32), 0.0)
        r = jax.nn.sigmoid(jnp.dot(conv, wir[...], preferred_element_type=_f32)
                           + bir[...]
                           + jnp.dot(ef, whr[...], preferred_element_type=_f32)
                           + bhr[...])
        z = jax.nn.sigmoid(jnp.dot(conv, wiz[...], preferred_element_type=_f32)
                           + biz[...]
                           + jnp.dot(ef, whz[...], preferred_element_type=_f32)
                           + bhz[...])
        n = jnp.tanh(jnp.dot(conv, win[...], preferred_element_type=_f32)
                     + bin_[...]
                     + r * (jnp.dot(ef, whn[...], preferred_element_type=_f32)
                            + bhn[...]))
        newef = (1.0 - z) * n + z * ef
        out_ref[...] = jnp.concatenate(
            [newef, jnp.zeros((newef.shape[0], out_f - H), _f32)], axis=1)

    consts = list(mw) + [rt] + list(gw)
    cspecs = [_full(w.shape) for w in consts]
    return pl.pallas_call(
        body,
        grid=(GE,),
        in_specs=[_blk((BE, F)), _blk((BE, 2 * H)), _blk((BE, 2 * H))] + cspecs,
        out_specs=_blk((BE, out_f)),
        out_shape=jax.ShapeDtypeStruct((E, out_f), _f32),
    )(g, ef16, ief16, *consts)


def _readout(partials):
    def body(a_ref, b_ref, out_ref):
        out_ref[...] = (a_ref[...] + b_ref[...])[:, :H]

    return pl.pallas_call(
        body,
        grid=(10,),
        in_specs=[
            pl.BlockSpec((NP // 10, F), lambda i: (i, 0)),
            pl.BlockSpec((NP // 10, F), lambda i: (i + 10, 0)),
        ],
        out_specs=pl.BlockSpec((NP // 10, H), lambda i: (i, 0)),
        out_shape=jax.ShapeDtypeStruct((NP, H), _f32),
    )(partials, partials)


def kernel(node_feats, edge_feats, edge_index, W_i, msg_W1, msg_b1, msg_W2,
           msg_b2, attn_W1, attn_b1, attn_W2, attn_b2, gru_Wih, gru_bih,
           gru_Whh, gru_bhh):
    src = edge_index[0]
    dst = edge_index[1]

    wab = jnp.concatenate([W_i[:128], W_i[128:256]], axis=1)
    wc = W_i[256:]
    rm = jnp.repeat(jnp.eye(H, dtype=_f32), H, axis=1)
    rt = rm.T
    mw = (msg_W1, msg_b1.reshape(1, H), msg_W2, msg_b2.reshape(1, H * H),
          attn_W1, attn_b1.reshape(1, H), attn_W2, attn_b2.reshape(1, H * H),
          rm)
    gw = (gru_Wih[:, :H], gru_Wih[:, H:2 * H], gru_Wih[:, 2 * H:],
          gru_Whh[:, :H], gru_Whh[:, H:2 * H], gru_Whh[:, 2 * H:],
          gru_bih[:H].reshape(1, H), gru_bih[H:2 * H].reshape(1, H),
          gru_bih[2 * H:].reshape(1, H),
          gru_bhh[:H].reshape(1, H), gru_bhh[H:2 * H].reshape(1, H),
          gru_bhh[2 * H:].reshape(1, H))

    t128 = _node_proj(node_feats, wab)
    g_s = _sc_gather(t128, src, F)
    g_d = _sc_gather(t128, dst, F)
    ef16 = _init_ef(g_s, g_d, edge_feats, wc)
    ief16 = ef16

    for step in range(3):
        payload = _pass1(ef16, mw)
        sm = _combine_partials(_sc_scatter(payload, dst, F), F)
        g = _sc_gather(sm, src, F)
        out_f = F if step == 2 else 2 * H
        newef = _pass2_gru(g, ef16, ief16, mw, rt, gw, out_f)
        if step < 2:
            ef16 = newef

    return _readout(_sc_scatter(newef, dst, F))[:N]


# speedup vs baseline: 46.2907x; 1.0969x over previous
import jax
import jax.numpy as jnp
from jax import lax
from jax.experimental import pallas as pl
from jax.experimental.pallas import tpu as pltpu
from jax.experimental.pallas import tpu_sc as plsc

N = 10000
NP = 10240
E = 160000
H = 8
F = 2 * H * H
NC = 2
NS = 16
NW = NC * NS
EPW = E // NW
CHX = 100
NITX = EPW // CHX
NB = 2
NPT = NP // NS

_f32 = jnp.float32
_MESH = dict(core_axis_name="c", subcore_axis_name="s")


def _sc_scatter(payload3, dstidx3, f):
    def body(p_hbm, idx_hbm, z_hbm, out_hbm, idx_sl, b0, b1,
             l0, l1, s0, s1, acc_sh):
        bufs = (b0, b1)
        lsems = (l0, l1)
        ssems = (s0, s1)
        c = lax.axis_index("c")
        s = lax.axis_index("s")
        wid = c * NS + s
        chunk0 = wid * NITX
        nsl = pl.ds(s * NPT, NPT)
        pltpu.sync_copy(z_hbm.at[nsl], acc_sh.at[nsl])
        pltpu.sync_copy(idx_hbm.at[wid], idx_sl)
        plsc.subcore_barrier()

        def load(i, b):
            pltpu.async_copy(p_hbm.at[chunk0 + i], bufs[b], lsems[b])

        for b in range(NB - 1):
            load(b, b)

        def outer(g, carry):
            i0 = g * NB
            for b in range(NB):
                i = i0 + b
                pltpu.make_async_copy(p_hbm.at[chunk0 + i], bufs[b],
                                      lsems[b]).wait()
                pltpu.async_copy(bufs[b], acc_sh.at[idx_sl.at[i]], ssems[b],
                                 add=True)
                nxt = i + NB - 1
                bn = (b + NB - 1) % NB

                @pl.when(nxt < NITX)
                def _():
                    @pl.when(i >= 1)
                    def _():
                        pltpu.make_async_copy(
                            bufs[bn], acc_sh.at[idx_sl.at[i - 1]],
                            ssems[bn]).wait()
                    load(nxt, bn)
            return carry

        lax.fori_loop(0, NITX // NB, outer, 0)
        for b in range(NB):
            i = NITX - NB + b
            pltpu.make_async_copy(bufs[b], acc_sh.at[idx_sl.at[i]],
                                  ssems[b]).wait()
        plsc.subcore_barrier()
        pltpu.sync_copy(acc_sh.at[nsl], out_hbm.at[pl.ds(c * NP + s * NPT, NPT)])

    zeros = jnp.zeros((NP, f), _f32)
    return pl.kernel(
        body,
        out_type=jax.ShapeDtypeStruct((2 * NP, f), _f32),
        mesh=plsc.VectorSubcoreMesh(**_MESH),
        scratch_types=(
            [pltpu.VMEM((NITX, CHX), jnp.int32)]
            + [pltpu.VMEM((CHX, f), _f32) for _ in range(NB)]
            + [pltpu.SemaphoreType.DMA] * (2 * NB)
            + [pltpu.VMEM_SHARED((NP, f), _f32)]
        ),
    )(payload3, dstidx3, zeros)


def _sc_gather(table, srcidx3, f):
    def body(t_hbm, idx_hbm, out_hbm, idx_sl, b0, b1,
             g0, g1, t0, t1, tbl_sh):
        bufs = (b0, b1)
        gsems = (g0, g1)
        stsems = (t0, t1)
        c = lax.axis_index("c")
        s = lax.axis_index("s")
        wid = c * NS + s
        chunk0 = wid * NITX
        nsl = pl.ds(s * NPT, NPT)
        pltpu.sync_copy(t_hbm.at[nsl], tbl_sh.at[nsl])
        pltpu.sync_copy(idx_hbm.at[wid], idx_sl)
        plsc.subcore_barrier()

        def gather(i, b):
            pltpu.async_copy(tbl_sh.at[idx_sl.at[i]], bufs[b], gsems[b])

        for b in range(NB - 1):
            gather(b, b)

        def outer(g, carry):
            i0 = g * NB
            for b in range(NB):
                i = i0 + b
                pltpu.make_async_copy(tbl_sh.at[idx_sl.at[i]], bufs[b],
                                      gsems[b]).wait()
                pltpu.async_copy(bufs[b], out_hbm.at[chunk0 + i], stsems[b])
                nxt = i + NB - 1
                bn = (b + NB - 1) % NB

                @pl.when(nxt < NITX)
                def _():
                    @pl.when(i >= 1)
                    def _():
                        pltpu.make_async_copy(
                            bufs[bn], out_hbm.at[chunk0 + i - 1],
                            stsems[bn]).wait()
                    gather(nxt, bn)
            return carry

        lax.fori_loop(0, NITX // NB, outer, 0)
        for b in range(NB):
            i = NITX - NB + b
            pltpu.make_async_copy(bufs[b], out_hbm.at[chunk0 + i],
                                  stsems[b]).wait()

    return pl.kernel(
        body,
        out_type=jax.ShapeDtypeStruct((NW * NITX, CHX, f), _f32),
        mesh=plsc.VectorSubcoreMesh(**_MESH),
        scratch_types=(
            [pltpu.VMEM((NITX, CHX), jnp.int32)]
            + [pltpu.VMEM((CHX, f), _f32) for _ in range(NB)]
            + [pltpu.SemaphoreType.DMA] * (2 * NB)
            + [pltpu.VMEM_SHARED((NP, f), _f32)]
        ),
    )(table, srcidx3)


BE = 8000
GE = E // BE


def _full(shape):
    nd = len(shape)
    return pl.BlockSpec(shape, lambda i: (0,) * nd)


def _blk(shape):
    return pl.BlockSpec(shape, lambda i: (i,) + (0,) * (len(shape) - 1))


def _node_proj(node_feats, wab):
    def body(nf_ref, w_ref, out_ref):
        ab = jnp.dot(nf_ref[...], w_ref[...], preferred_element_type=_f32)
        out_ref[...] = jnp.concatenate(
            [ab, jnp.zeros((ab.shape[0], F - 2 * H), _f32)], axis=1)

    return pl.pallas_call(
        body,
        grid=(10,),
        in_specs=[_blk((N // 10, 128)), _full((128, 2 * H))],
        out_specs=_blk((N // 10, F)),
        out_shape=jax.ShapeDtypeStruct((NP, F), _f32),
    )(node_feats, wab)


def _init_ef(g_s, g_d, edge_feats, wc):
    def body(s_ref, d_ref, ef_ref, w_ref, out_ref):
        x = (s_ref[...][:, :H] + d_ref[...][:, H:2 * H]
             + jnp.dot(ef_ref[...], w_ref[...], preferred_element_type=_f32))
        x = jnp.maximum(x, 0.0)
        out_ref[...] = jnp.concatenate([x, jnp.zeros_like(x)], axis=1)

    return pl.pallas_call(
        body,
        grid=(GE,),
        in_specs=[_blk((BE, F)), _blk((BE, F)), _blk((BE, 16)),
                  _full((16, H))],
        out_specs=_blk((BE, 2 * H)),
        out_shape=jax.ShapeDtypeStruct((E, 2 * H), _f32),
    )(g_s, g_d, edge_feats, wc)


def _edge_weights(ef, w1m, b1m, w2m, b2m, w1a, b1a, w2a, b2a, rm):
    t_m = jnp.maximum(jnp.dot(ef, w1m, preferred_element_type=_f32) + b1m, 0.0)
    w_m = jnp.dot(t_m, w2m, preferred_element_type=_f32) + b2m
    t_a = jnp.maximum(jnp.dot(ef, w1a, preferred_element_type=_f32) + b1a, 0.0)
    w_a = jnp.dot(t_a, w2a, preferred_element_type=_f32) + b2a
    ef_r = jnp.dot(ef, rm, preferred_element_type=_f32)
    return w_m, w_a, ef_r


def _pass1(ef16, mw):
    def body(ef_ref, w1m, b1m, w2m, b2m, w1a, b1a, w2a, b2a, rm, out_ref):
        ef = ef_ref[...][:, :H]
        w_m, w_a, ef_r = _edge_weights(ef, w1m[...], b1m[...], w2m[...],
                                       b2m[...], w1a[...], b1a[...],
                                       w2a[...], b2a[...], rm[...])
        exp_e2 = jnp.exp(w_a * ef_r)
        h1 = exp_e2 * (w_m * ef_r)
        out_ref[...] = jnp.concatenate([exp_e2, h1], axis=1)

    wspecs = [_full(w.shape) for w in mw]
    return pl.pallas_call(
        body,
        grid=(GE,),
        in_specs=[_blk((BE, 2 * H))] + wspecs,
        out_specs=_blk((BE, F)),
        out_shape=jax.ShapeDtypeStruct((E, F), _f32),
    )(ef16, *mw)


def _combine_partials(partials, f):
    def body(a_ref, b_ref, out_ref):
        out_ref[...] = a_ref[...] + b_ref[...]

    return pl.pallas_call(
        body,
        grid=(10,),
        in_specs=[
            pl.BlockSpec((NP // 10, f), lambda i: (i, 0)),
            pl.BlockSpec((NP // 10, f), lambda i: (i + 10, 0)),
        ],
        out_specs=pl.BlockSpec((NP // 10, f), lambda i: (i, 0)),
        out_shape=jax.ShapeDtypeStruct((NP, f), _f32),
    )(partials, partials)


def _pass2_gru(g, ef16, ief16, mw, rt, gw, out_f=2 * H):
    def body(g_ref, ef_ref, ief_ref, w1m, b1m, w2m, b2m, w1a, b1a, w2a, b2a,
             rm, rt_ref, wir, wiz, win, whr, whz, whn, bir, biz, bin_,
             bhr, bhz, bhn, out_ref):
        ef = ef_ref[...][:, :H]
        ief = ief_ref[...][:, :H]
        w_m, w_a, ef_r = _edge_weights(ef, w1m[...], b1m[...], w2m[...],
                                       b2m[...], w1a[...], b1a[...],
                                       w2a[...], b2a[...], rm[...])
        ief_r = jnp.dot(ief, rm[...], preferred_element_type=_f32)
        exp_e2 = jnp.exp(w_a * ef_r)
        h1 = exp_e2 * (w_m * ef_r)
        exp_ie2 = jnp.exp(w_a * ief_r)
        ih1 = exp_ie2 * (w_m * ief_r)
        gathered = g_ref[...]
        sg = gathered[:, :H * H]
        mg = gathered[:, H * H:]
        h2 = (mg - h1 + ih1) / (sg - exp_e2 + exp_ie2)
        conv = jnp.maximum(jnp.dot(h2, rt_ref[...],
                                   preferred_element_type=_f32), 0.0)
        r = jax.nn.sigmoid(jnp.dot(conv, wir[...], preferred_element_type=_f32)
                           + bir[...]
                           + jnp.dot(ef, whr[...], preferred_element_type=_f32)
                           + bhr[...])
        z = jax.nn.sigmoid(jnp.dot(conv, wiz[...], preferred_element_type=_f32)
                           + biz[...]
                           + jnp.dot(ef, whz[...], preferred_element_type=_f32)
                           + bhz[...])
        n = jnp.tanh(jnp.dot(conv, win[...], preferred_element_type=_f32)
                     + bin_[...]
                     + r * (jnp.dot(ef, whn[...], preferred_element_type=_f32)
                            + bhn[...]))
        newef = (1.0 - z) * n + z * ef
        out_ref[...] = jnp.concatenate(
            [newef, jnp.zeros((newef.shape[0], out_f - H), _f32)], axis=1)

    consts = list(mw) + [rt] + list(gw)
    cspecs = [_full(w.shape) for w in consts]
    return pl.pallas_call(
        body,
        grid=(GE,),
        in_specs=[_blk((BE, F)), _blk((BE, 2 * H)), _blk((BE, 2 * H))] + cspecs,
        out_specs=_blk((BE, out_f)),
        out_shape=jax.ShapeDtypeStruct((E, out_f), _f32),
    )(g, ef16, ief16, *consts)


def _readout(partials):
    def body(a_ref, b_ref, out_ref):
        out_ref[...] = (a_ref[...] + b_ref[...])[:, :H]

    return pl.pallas_call(
        body,
        grid=(10,),
        in_specs=[
            pl.BlockSpec((NP // 10, F), lambda i: (i, 0)),
            pl.BlockSpec((NP // 10, F), lambda i: (i + 10, 0)),
        ],
        out_specs=pl.BlockSpec((NP // 10, H), lambda i: (i, 0)),
        out_shape=jax.ShapeDtypeStruct((NP, H), _f32),
    )(partials, partials)


def kernel(node_feats, edge_feats, edge_index, W_i, msg_W1, msg_b1, msg_W2,
           msg_b2, attn_W1, attn_b1, attn_W2, attn_b2, gru_Wih, gru_bih,
           gru_Whh, gru_bhh):
    src3 = edge_index[0].reshape(NW, NITX, CHX)
    dst3 = edge_index[1].reshape(NW, NITX, CHX)

    wab = jnp.concatenate([W_i[:128], W_i[128:256]], axis=1)
    wc = W_i[256:]
    rm = jnp.repeat(jnp.eye(H, dtype=_f32), H, axis=1)
    rt = rm.T
    mw = (msg_W1, msg_b1.reshape(1, H), msg_W2, msg_b2.reshape(1, H * H),
          attn_W1, attn_b1.reshape(1, H), attn_W2, attn_b2.reshape(1, H * H),
          rm)
    gw = (gru_Wih[:, :H], gru_Wih[:, H:2 * H], gru_Wih[:, 2 * H:],
          gru_Whh[:, :H], gru_Whh[:, H:2 * H], gru_Whh[:, 2 * H:],
          gru_bih[:H].reshape(1, H), gru_bih[H:2 * H].reshape(1, H),
          gru_bih[2 * H:].reshape(1, H),
          gru_bhh[:H].reshape(1, H), gru_bhh[H:2 * H].reshape(1, H),
          gru_bhh[2 * H:].reshape(1, H))

    t128 = _node_proj(node_feats, wab)
    g_s = _sc_gather(t128, src3, F).reshape(E, F)
    g_d = _sc_gather(t128, dst3, F).reshape(E, F)
    ef16 = _init_ef(g_s, g_d, edge_feats, wc)
    ief16 = ef16

    for step in range(3):
        payload = _pass1(ef16, mw).reshape(NW * NITX, CHX, F)
        sm = _combine_partials(_sc_scatter(payload, dst3, F), F)
        g = _sc_gather(sm, src3, F).reshape(E, F)
        out_f = F if step == 2 else 2 * H
        newef = _pass2_gru(g, ef16, ief16, mw, rt, gw, out_f)
        if step < 2:
            ef16 = newef

    final = newef.reshape(NW * NITX, CHX, F)
    return _readout(_sc_scatter(final, dst3, F))[:N]


# speedup vs baseline: 48.4588x; 1.0468x over previous
import jax
import jax.numpy as jnp
from jax import lax
from jax.experimental import pallas as pl
from jax.experimental.pallas import tpu as pltpu
from jax.experimental.pallas import tpu_sc as plsc

N = 10000
NP = 10240
E = 160000
H = 8
F = 2 * H * H
NC = 2
NS = 16
NW = NC * NS
EPW = E // NW
CHX = 100
NITX = EPW // CHX
NB = 2
NPT = NP // NS

_f32 = jnp.float32
_MESH = dict(core_axis_name="c", subcore_axis_name="s")


def _sc_scatter(payload3, dstidx3, f):
    def body(p_hbm, idx_hbm, z_hbm, out_hbm, idx_sl, b0, b1,
             l0, l1, s0, s1, acc_sh):
        bufs = (b0, b1)
        lsems = (l0, l1)
        ssems = (s0, s1)
        c = lax.axis_index("c")
        s = lax.axis_index("s")
        wid = c * NS + s
        chunk0 = wid * NITX
        nsl = pl.ds(s * NPT, NPT)
        pltpu.sync_copy(z_hbm.at[nsl], acc_sh.at[nsl])
        pltpu.sync_copy(idx_hbm.at[wid], idx_sl)
        plsc.subcore_barrier()

        def load(i, b):
            pltpu.async_copy(p_hbm.at[chunk0 + i], bufs[b], lsems[b])

        for b in range(NB - 1):
            load(b, b)

        def outer(g, carry):
            i0 = g * NB
            for b in range(NB):
                i = i0 + b
                pltpu.make_async_copy(p_hbm.at[chunk0 + i], bufs[b],
                                      lsems[b]).wait()
                pltpu.async_copy(bufs[b], acc_sh.at[idx_sl.at[i]], ssems[b],
                                 add=True)
                nxt = i + NB - 1
                bn = (b + NB - 1) % NB

                @pl.when(nxt < NITX)
                def _():
                    @pl.when(i >= 1)
                    def _():
                        pltpu.make_async_copy(
                            bufs[bn], acc_sh.at[idx_sl.at[i - 1]],
                            ssems[bn]).wait()
                    load(nxt, bn)
            return carry

        lax.fori_loop(0, NITX // NB, outer, 0)
        for b in range(NB):
            i = NITX - NB + b
            pltpu.make_async_copy(bufs[b], acc_sh.at[idx_sl.at[i]],
                                  ssems[b]).wait()
        plsc.subcore_barrier()
        pltpu.sync_copy(acc_sh.at[nsl], out_hbm.at[pl.ds(c * NP + s * NPT, NPT)])

    zeros = jnp.zeros((NP, f), _f32)
    return pl.kernel(
        body,
        out_type=jax.ShapeDtypeStruct((2 * NP, f), _f32),
        mesh=plsc.VectorSubcoreMesh(**_MESH),
        scratch_types=(
            [pltpu.VMEM((NITX, CHX), jnp.int32)]
            + [pltpu.VMEM((CHX, f), _f32) for _ in range(NB)]
            + [pltpu.SemaphoreType.DMA] * (2 * NB)
            + [pltpu.VMEM_SHARED((NP, f), _f32)]
        ),
    )(payload3, dstidx3, zeros)


def _sc_gather(table, srcidx3, f):
    def body(t_hbm, idx_hbm, out_hbm, idx_sl, b0, b1,
             g0, g1, t0, t1, tbl_sh):
        bufs = (b0, b1)
        gsems = (g0, g1)
        stsems = (t0, t1)
        c = lax.axis_index("c")
        s = lax.axis_index("s")
        wid = c * NS + s
        chunk0 = wid * NITX
        nsl = pl.ds(s * NPT, NPT)
        pltpu.sync_copy(t_hbm.at[nsl], tbl_sh.at[nsl])
        pltpu.sync_copy(idx_hbm.at[wid], idx_sl)
        plsc.subcore_barrier()

        def gather(i, b):
            pltpu.async_copy(tbl_sh.at[idx_sl.at[i]], bufs[b], gsems[b])

        for b in range(NB - 1):
            gather(b, b)

        def outer(g, carry):
            i0 = g * NB
            for b in range(NB):
                i = i0 + b
                pltpu.make_async_copy(tbl_sh.at[idx_sl.at[i]], bufs[b],
                                      gsems[b]).wait()
                pltpu.async_copy(bufs[b], out_hbm.at[chunk0 + i], stsems[b])
                nxt = i + NB - 1
                bn = (b + NB - 1) % NB

                @pl.when(nxt < NITX)
                def _():
                    @pl.when(i >= 1)
                    def _():
                        pltpu.make_async_copy(
                            bufs[bn], out_hbm.at[chunk0 + i - 1],
                            stsems[bn]).wait()
                    gather(nxt, bn)
            return carry

        lax.fori_loop(0, NITX // NB, outer, 0)
        for b in range(NB):
            i = NITX - NB + b
            pltpu.make_async_copy(bufs[b], out_hbm.at[chunk0 + i],
                                  stsems[b]).wait()

    return pl.kernel(
        body,
        out_type=jax.ShapeDtypeStruct((NW * NITX, CHX, f), _f32),
        mesh=plsc.VectorSubcoreMesh(**_MESH),
        scratch_types=(
            [pltpu.VMEM((NITX, CHX), jnp.int32)]
            + [pltpu.VMEM((CHX, f), _f32) for _ in range(NB)]
            + [pltpu.SemaphoreType.DMA] * (2 * NB)
            + [pltpu.VMEM_SHARED((NP, f), _f32)]
        ),
    )(table, srcidx3)


BE = 2000
GE = E // BE
CPB = BE // CHX


def _full(shape):
    nd = len(shape)
    return pl.BlockSpec(shape, lambda i: (0,) * nd)


def _blk(shape):
    return pl.BlockSpec(shape, lambda i: (i,) + (0,) * (len(shape) - 1))


def _node_proj(node_feats, wab):
    def body(nf_ref, w_ref, out_ref):
        ab = jnp.dot(nf_ref[...], w_ref[...], preferred_element_type=_f32)
        out_ref[...] = jnp.concatenate(
            [ab, jnp.zeros((ab.shape[0], F - 2 * H), _f32)], axis=1)

    return pl.pallas_call(
        body,
        grid=(10,),
        in_specs=[_blk((N // 10, 128)), _full((128, 2 * H))],
        out_specs=_blk((N // 10, F)),
        out_shape=jax.ShapeDtypeStruct((NP, F), _f32),
    )(node_feats, wab)


def _init_ef(g_s, g_d, edge_feats, wc):
    def body(s_ref, d_ref, ef_ref, w_ref, out_ref):
        gs = s_ref[...].reshape(BE, F)
        gd = d_ref[...].reshape(BE, F)
        x = (gs[:, :H] + gd[:, H:2 * H]
             + jnp.dot(ef_ref[...], w_ref[...], preferred_element_type=_f32))
        x = jnp.maximum(x, 0.0)
        out_ref[...] = jnp.concatenate([x, jnp.zeros_like(x)], axis=1)

    return pl.pallas_call(
        body,
        grid=(GE,),
        in_specs=[_blk((CPB, CHX, F)), _blk((CPB, CHX, F)), _blk((BE, 16)),
                  _full((16, H))],
        out_specs=_blk((BE, 2 * H)),
        out_shape=jax.ShapeDtypeStruct((E, 2 * H), _f32),
    )(g_s, g_d, edge_feats, wc)


def _edge_weights(ef, c1, b1, w2blk, b2):
    t = jnp.dot(ef, c1, preferred_element_type=_f32) + b1
    u = jnp.maximum(t[:, :2 * H], 0.0)
    wma = jnp.dot(u, w2blk, preferred_element_type=_f32) + b2
    ef_r = t[:, 2 * H:2 * H + H * H]
    return wma[:, :H * H], wma[:, H * H:], ef_r, t


def _pass1(ef16, c1, b1, w2blk, b2):
    def body(ef_ref, c1_ref, b1_ref, w2_ref, b2_ref, out_ref):
        ef = ef_ref[...][:, :H]
        w_m, w_a, ef_r, _ = _edge_weights(ef, c1_ref[...], b1_ref[...],
                                          w2_ref[...], b2_ref[...])
        exp_e2 = jnp.exp(w_a * ef_r)
        h1 = exp_e2 * (w_m * ef_r)
        out_ref[...] = jnp.concatenate([exp_e2, h1],
                                       axis=1).reshape(CPB, CHX, F)

    return pl.pallas_call(
        body,
        grid=(GE,),
        in_specs=[_blk((BE, 2 * H)), _full((H, 10 * H)), _full((1, 10 * H)),
                  _full((2 * H, F)), _full((1, F))],
        out_specs=_blk((CPB, CHX, F)),
        out_shape=jax.ShapeDtypeStruct((NW * NITX, CHX, F), _f32),
    )(ef16, c1, b1, w2blk, b2)


def _combine_partials(partials, f):
    def body(a_ref, b_ref, out_ref):
        out_ref[...] = a_ref[...] + b_ref[...]

    return pl.pallas_call(
        body,
        grid=(10,),
        in_specs=[
            pl.BlockSpec((NP // 10, f), lambda i: (i, 0)),
            pl.BlockSpec((NP // 10, f), lambda i: (i + 10, 0)),
        ],
        out_specs=pl.BlockSpec((NP // 10, f), lambda i: (i, 0)),
        out_shape=jax.ShapeDtypeStruct((NP, f), _f32),
    )(partials, partials)


def _pass2_gru(g, ef16, ief16, c2, b2c, w2blk, b2, rm, rt, wih, bih,
               out_3d=False):
    def body(g_ref, ef_ref, ief_ref, c2_ref, b2c_ref, w2_ref, b2_ref, rm_ref,
             rt_ref, wih_ref, bih_ref, out_ref):
        ef = ef_ref[...][:, :H]
        ief = ief_ref[...][:, :H]
        w_m, w_a, ef_r, t = _edge_weights(ef, c2_ref[...], b2c_ref[...],
                                          w2_ref[...], b2_ref[...])
        gh = t[:, 10 * H:13 * H]
        ief_r = jnp.dot(ief, rm_ref[...], preferred_element_type=_f32)
        exp_e2 = jnp.exp(w_a * ef_r)
        h1 = exp_e2 * (w_m * ef_r)
        exp_ie2 = jnp.exp(w_a * ief_r)
        ih1 = exp_ie2 * (w_m * ief_r)
        gathered = g_ref[...].reshape(BE, F)
        sg = gathered[:, :H * H]
        mg = gathered[:, H * H:]
        h2 = (mg - h1 + ih1) / (sg - exp_e2 + exp_ie2)
        conv = jnp.maximum(jnp.dot(h2, rt_ref[...],
                                   preferred_element_type=_f32), 0.0)
        gi = jnp.dot(conv, wih_ref[...], preferred_element_type=_f32) + bih_ref[...]
        r = jax.nn.sigmoid(gi[:, :H] + gh[:, :H])
        z = jax.nn.sigmoid(gi[:, H:2 * H] + gh[:, H:2 * H])
        n = jnp.tanh(gi[:, 2 * H:] + r * gh[:, 2 * H:])
        newef = (1.0 - z) * n + z * ef
        if out_3d:
            out_ref[...] = jnp.concatenate(
                [newef, jnp.zeros((BE, F - H), _f32)],
                axis=1).reshape(CPB, CHX, F)
        else:
            out_ref[...] = jnp.concatenate([newef, jnp.zeros_like(newef)],
                                           axis=1)

    if out_3d:
        out_spec = _blk((CPB, CHX, F))
        out_shape = jax.ShapeDtypeStruct((NW * NITX, CHX, F), _f32)
    else:
        out_spec = _blk((BE, 2 * H))
        out_shape = jax.ShapeDtypeStruct((E, 2 * H), _f32)
    return pl.pallas_call(
        body,
        grid=(GE,),
        in_specs=[_blk((CPB, CHX, F)), _blk((BE, 2 * H)), _blk((BE, 2 * H)),
                  _full((H, 13 * H)), _full((1, 13 * H)), _full((2 * H, F)),
                  _full((1, F)), _full((H, H * H)), _full((H * H, H)),
                  _full((H, 3 * H)), _full((1, 3 * H))],
        out_specs=out_spec,
        out_shape=out_shape,
    )(g, ef16, ief16, c2, b2c, w2blk, b2, rm, rt, wih, bih)


def _readout(partials):
    def body(a_ref, b_ref, out_ref):
        out_ref[...] = (a_ref[...] + b_ref[...])[:, :H]

    return pl.pallas_call(
        body,
        grid=(10,),
        in_specs=[
            pl.BlockSpec((NP // 10, F), lambda i: (i, 0)),
            pl.BlockSpec((NP // 10, F), lambda i: (i + 10, 0)),
        ],
        out_specs=pl.BlockSpec((NP // 10, H), lambda i: (i, 0)),
        out_shape=jax.ShapeDtypeStruct((NP, H), _f32),
    )(partials, partials)


def kernel(node_feats, edge_feats, edge_index, W_i, msg_W1, msg_b1, msg_W2,
           msg_b2, attn_W1, attn_b1, attn_W2, attn_b2, gru_Wih, gru_bih,
           gru_Whh, gru_bhh):
    src3 = edge_index[0].reshape(NW, NITX, CHX)
    dst3 = edge_index[1].reshape(NW, NITX, CHX)

    wab = jnp.concatenate([W_i[:128], W_i[128:256]], axis=1)
    wc = W_i[256:]
    rm = jnp.repeat(jnp.eye(H, dtype=_f32), H, axis=1)
    rt = rm.T
    zH = jnp.zeros((H, H), _f32)
    c1 = jnp.concatenate([msg_W1, attn_W1, rm], axis=1)
    b1 = jnp.concatenate([msg_b1, attn_b1, jnp.zeros((H * H,), _f32)]
                         ).reshape(1, 10 * H)
    w2blk = jnp.concatenate([
        jnp.concatenate([msg_W2, jnp.zeros((H, H * H), _f32)], axis=1),
        jnp.concatenate([jnp.zeros((H, H * H), _f32), attn_W2], axis=1),
    ], axis=0)
    b2 = jnp.concatenate([msg_b2, attn_b2]).reshape(1, F)
    c2 = jnp.concatenate([msg_W1, attn_W1, rm, gru_Whh], axis=1)
    b2c = jnp.concatenate([msg_b1, attn_b1, jnp.zeros((H * H,), _f32),
                           gru_bhh]).reshape(1, 13 * H)
    wih = gru_Wih
    bih = gru_bih.reshape(1, 3 * H)

    t128 = _node_proj(node_feats, wab)
    g_s = _sc_gather(t128, src3, F)
    g_d = _sc_gather(t128, dst3, F)
    ef16 = _init_ef(g_s, g_d, edge_feats, wc)
    ief16 = ef16

    for step in range(3):
        payload = _pass1(ef16, c1, b1, w2blk, b2)
        sm = _combine_partials(_sc_scatter(payload, dst3, F), F)
        g = _sc_gather(sm, src3, F)
        newef = _pass2_gru(g, ef16, ief16, c2, b2c, w2blk, b2, rm, rt,
                           wih, bih, out_3d=(step == 2))
        if step < 2:
            ef16 = newef

    return _readout(_sc_scatter(newef, dst3, F))[:N]


# speedup vs baseline: 55.9178x; 1.1539x over previous
import jax
import jax.numpy as jnp
from jax import lax
from jax.experimental import pallas as pl
from jax.experimental.pallas import tpu as pltpu
from jax.experimental.pallas import tpu_sc as plsc

N = 10000
NP = 10240
E = 160000
H = 8
F = 2 * H * H
NC = 2
NS = 16
NW = NC * NS
EPW = E // NW
CHX = 100
NITX = EPW // CHX
NB = 2
NPT = NP // NS

_f32 = jnp.float32
_MESH = dict(core_axis_name="c", subcore_axis_name="s")


def _sc_scatter(payload3, dstidx3, f):
    def body(p_hbm, idx_hbm, z_hbm, out_hbm, idx_sl, b0, b1,
             l0, l1, s0, s1, acc_sh):
        bufs = (b0, b1)
        lsems = (l0, l1)
        ssems = (s0, s1)
        c = lax.axis_index("c")
        s = lax.axis_index("s")
        wid = c * NS + s
        chunk0 = wid * NITX
        nsl = pl.ds(s * NPT, NPT)
        pltpu.sync_copy(z_hbm.at[nsl], acc_sh.at[nsl])
        pltpu.sync_copy(idx_hbm.at[wid], idx_sl)
        plsc.subcore_barrier()

        def load(i, b):
            pltpu.async_copy(p_hbm.at[chunk0 + i], bufs[b], lsems[b])

        for b in range(NB - 1):
            load(b, b)

        def outer(g, carry):
            i0 = g * NB
            for b in range(NB):
                i = i0 + b
                pltpu.make_async_copy(p_hbm.at[chunk0 + i], bufs[b],
                                      lsems[b]).wait()
                pltpu.async_copy(bufs[b], acc_sh.at[idx_sl.at[i]], ssems[b],
                                 add=True)
                nxt = i + NB - 1
                bn = (b + NB - 1) % NB

                @pl.when(nxt < NITX)
                def _():
                    @pl.when(i >= 1)
                    def _():
                        pltpu.make_async_copy(
                            bufs[bn], acc_sh.at[idx_sl.at[i - 1]],
                            ssems[bn]).wait()
                    load(nxt, bn)
            return carry

        lax.fori_loop(0, NITX // NB, outer, 0)
        for b in range(NB):
            i = NITX - NB + b
            pltpu.make_async_copy(bufs[b], acc_sh.at[idx_sl.at[i]],
                                  ssems[b]).wait()
        plsc.subcore_barrier()
        pltpu.sync_copy(acc_sh.at[nsl], out_hbm.at[pl.ds(c * NP + s * NPT, NPT)])

    zeros = jnp.zeros((NP, f), _f32)
    return pl.kernel(
        body,
        out_type=jax.ShapeDtypeStruct((2 * NP, f), _f32),
        mesh=plsc.VectorSubcoreMesh(**_MESH),
        scratch_types=(
            [pltpu.VMEM((NITX, CHX), jnp.int32)]
            + [pltpu.VMEM((CHX, f), _f32) for _ in range(NB)]
            + [pltpu.SemaphoreType.DMA] * (2 * NB)
            + [pltpu.VMEM_SHARED((NP, f), _f32)]
        ),
    )(payload3, dstidx3, zeros)


def _sc_gather(table, srcidx3, f):
    def body(t_hbm, idx_hbm, out_hbm, idx_sl, b0, b1,
             g0, g1, t0, t1, tbl_sh):
        bufs = (b0, b1)
        gsems = (g0, g1)
        stsems = (t0, t1)
        c = lax.axis_index("c")
        s = lax.axis_index("s")
        wid = c * NS + s
        chunk0 = wid * NITX
        nsl = pl.ds(s * NPT, NPT)
        pltpu.sync_copy(t_hbm.at[nsl], tbl_sh.at[nsl])
        pltpu.sync_copy(idx_hbm.at[wid], idx_sl)
        plsc.subcore_barrier()

        def gather(i, b):
            pltpu.async_copy(tbl_sh.at[idx_sl.at[i]], bufs[b], gsems[b])

        for b in range(NB - 1):
            gather(b, b)

        def outer(g, carry):
            i0 = g * NB
            for b in range(NB):
                i = i0 + b
                pltpu.make_async_copy(tbl_sh.at[idx_sl.at[i]], bufs[b],
                                      gsems[b]).wait()
                pltpu.async_copy(bufs[b], out_hbm.at[chunk0 + i], stsems[b])
                nxt = i + NB - 1
                bn = (b + NB - 1) % NB

                @pl.when(nxt < NITX)
                def _():
                    @pl.when(i >= 1)
                    def _():
                        pltpu.make_async_copy(
                            bufs[bn], out_hbm.at[chunk0 + i - 1],
                            stsems[bn]).wait()
                    gather(nxt, bn)
            return carry

        lax.fori_loop(0, NITX // NB, outer, 0)
        for b in range(NB):
            i = NITX - NB + b
            pltpu.make_async_copy(bufs[b], out_hbm.at[chunk0 + i],
                                  stsems[b]).wait()

    return pl.kernel(
        body,
        out_type=jax.ShapeDtypeStruct((NW * NITX, CHX, f), _f32),
        mesh=plsc.VectorSubcoreMesh(**_MESH),
        scratch_types=(
            [pltpu.VMEM((NITX, CHX), jnp.int32)]
            + [pltpu.VMEM((CHX, f), _f32) for _ in range(NB)]
            + [pltpu.SemaphoreType.DMA] * (2 * NB)
            + [pltpu.VMEM_SHARED((NP, f), _f32)]
        ),
    )(table, srcidx3)


BE = 2000
GE = E // BE
CPB = BE // CHX


def _full(shape):
    nd = len(shape)
    return pl.BlockSpec(shape, lambda i: (0,) * nd)


def _blk(shape):
    return pl.BlockSpec(shape, lambda i: (i,) + (0,) * (len(shape) - 1))


def _node_proj(node_feats, wab):
    def body(nf_ref, w_ref, out_ref):
        ab = jnp.dot(nf_ref[...], w_ref[...], preferred_element_type=_f32)
        out_ref[...] = jnp.concatenate(
            [ab, jnp.zeros((ab.shape[0], F - 2 * H), _f32)], axis=1)

    return pl.pallas_call(
        body,
        grid=(10,),
        in_specs=[_blk((N // 10, 128)), _full((128, 2 * H))],
        out_specs=_blk((N // 10, F)),
        out_shape=jax.ShapeDtypeStruct((NP, F), _f32),
    )(node_feats, wab)


def _init_ef(g_s, g_d, edge_feats, wc):
    def body(s_ref, d_ref, ef_ref, w_ref, out_ref):
        gs = s_ref[...].reshape(BE, F)
        gd = d_ref[...].reshape(BE, F)
        x = (gs[:, :H] + gd[:, H:2 * H]
             + jnp.dot(ef_ref[...], w_ref[...], preferred_element_type=_f32))
        x = jnp.maximum(x, 0.0)
        out_ref[...] = jnp.concatenate([x, jnp.zeros_like(x)], axis=1)

    return pl.pallas_call(
        body,
        grid=(GE,),
        in_specs=[_blk((CPB, CHX, F)), _blk((CPB, CHX, F)), _blk((BE, 16)),
                  _full((16, H))],
        out_specs=_blk((BE, 2 * H)),
        out_shape=jax.ShapeDtypeStruct((E, 2 * H), _f32),
    )(g_s, g_d, edge_feats, wc)


def _edge_mats(ef16, w1cat, b1cat, w2blk, b2cat, rm2):
    t = jnp.dot(ef16, w1cat, preferred_element_type=_f32) + b1cat
    u = jnp.maximum(t, 0.0)
    wma = jnp.dot(u, w2blk, preferred_element_type=_f32) + b2cat
    ef_r2 = jnp.dot(ef16, rm2, preferred_element_type=_f32)
    return wma, wma * ef_r2


def _pass1(ef16, w1cat, b1cat, w2blk, b2cat, rm2):
    def body(ef_ref, w1_ref, b1_ref, w2_ref, b2_ref, rm2_ref, out_ref):
        _, e_all = _edge_mats(ef_ref[...], w1_ref[...], b1_ref[...],
                              w2_ref[...], b2_ref[...], rm2_ref[...])
        exp_e2 = jnp.exp(e_all[:, H * H:])
        h1 = exp_e2 * e_all[:, :H * H]
        out_ref[...] = jnp.concatenate([exp_e2, h1],
                                       axis=1).reshape(CPB, CHX, F)

    return pl.pallas_call(
        body,
        grid=(GE,),
        in_specs=[_blk((BE, 2 * H)), _full((2 * H, 2 * H)), _full((1, 2 * H)),
                  _full((2 * H, F)), _full((1, F)), _full((2 * H, F))],
        out_specs=_blk((CPB, CHX, F)),
        out_shape=jax.ShapeDtypeStruct((NW * NITX, CHX, F), _f32),
    )(ef16, w1cat, b1cat, w2blk, b2cat, rm2)


def _combine_partials(partials, f):
    def body(a_ref, b_ref, out_ref):
        out_ref[...] = a_ref[...] + b_ref[...]

    return pl.pallas_call(
        body,
        grid=(10,),
        in_specs=[
            pl.BlockSpec((NP // 10, f), lambda i: (i, 0)),
            pl.BlockSpec((NP // 10, f), lambda i: (i + 10, 0)),
        ],
        out_specs=pl.BlockSpec((NP // 10, f), lambda i: (i, 0)),
        out_shape=jax.ShapeDtypeStruct((NP, f), _f32),
    )(partials, partials)


def _pass2_gru(g, ef16, ief16, cw, out_3d=False):
    def body(g_ref, ef_ref, ief_ref, w1_ref, b1_ref, w2_ref, b2_ref, rm2_ref,
             rt_ref, wir_ref, wiz_ref, win_ref, whr_ref, whz_ref, whn_ref,
             gb_ref, out_ref):
        ef16v = ef_ref[...]
        wma, e_all = _edge_mats(ef16v, w1_ref[...], b1_ref[...], w2_ref[...],
                                b2_ref[...], rm2_ref[...])
        ie_all = wma * jnp.dot(ief_ref[...], rm2_ref[...],
                               preferred_element_type=_f32)
        exp_e2 = jnp.exp(e_all[:, H * H:])
        h1 = exp_e2 * e_all[:, :H * H]
        exp_ie2 = jnp.exp(ie_all[:, H * H:])
        ih1 = exp_ie2 * ie_all[:, :H * H]
        gathered = g_ref[...].reshape(BE, F)
        sg = gathered[:, :H * H]
        mg = gathered[:, H * H:]
        h2 = (mg - h1 + ih1) / (sg - exp_e2 + exp_ie2)
        conv = jnp.maximum(jnp.dot(h2, rt_ref[...],
                                   preferred_element_type=_f32), 0.0)
        gb = gb_ref[...]
        r = jax.nn.sigmoid(
            jnp.dot(conv, wir_ref[...], preferred_element_type=_f32)
            + jnp.dot(ef16v, whr_ref[...], preferred_element_type=_f32)
            + gb[:, :H])
        z = jax.nn.sigmoid(
            jnp.dot(conv, wiz_ref[...], preferred_element_type=_f32)
            + jnp.dot(ef16v, whz_ref[...], preferred_element_type=_f32)
            + gb[:, H:2 * H])
        n = jnp.tanh(
            jnp.dot(conv, win_ref[...], preferred_element_type=_f32)
            + gb[:, 2 * H:3 * H]
            + r * (jnp.dot(ef16v, whn_ref[...], preferred_element_type=_f32)
                   + gb[:, 3 * H:]))
        newef = (1.0 - z) * n + z * ef16v[:, :H]
        if out_3d:
            out_ref[...] = jnp.concatenate(
                [newef, jnp.zeros((BE, F - H), _f32)],
                axis=1).reshape(CPB, CHX, F)
        else:
            out_ref[...] = jnp.concatenate([newef, jnp.zeros_like(newef)],
                                           axis=1)

    if out_3d:
        out_spec = _blk((CPB, CHX, F))
        out_shape = jax.ShapeDtypeStruct((NW * NITX, CHX, F), _f32)
    else:
        out_spec = _blk((BE, 2 * H))
        out_shape = jax.ShapeDtypeStruct((E, 2 * H), _f32)
    (w1cat, b1cat, w2blk, b2cat, rm2, rt, wir, wiz, win, whr, whz, whn,
     gbias) = cw
    return pl.pallas_call(
        body,
        grid=(GE,),
        in_specs=[_blk((CPB, CHX, F)), _blk((BE, 2 * H)), _blk((BE, 2 * H)),
                  _full((2 * H, 2 * H)), _full((1, 2 * H)), _full((2 * H, F)),
                  _full((1, F)), _full((2 * H, F)), _full((H * H, H)),
                  _full((H, H)), _full((H, H)), _full((H, H)),
                  _full((2 * H, H)), _full((2 * H, H)), _full((2 * H, H)),
                  _full((1, 4 * H))],
        out_specs=out_spec,
        out_shape=out_shape,
    )(g, ef16, ief16, w1cat, b1cat, w2blk, b2cat, rm2, rt, wir, wiz, win,
      whr, whz, whn, gbias)


def _readout(partials):
    def body(a_ref, b_ref, out_ref):
        out_ref[...] = (a_ref[...] + b_ref[...])[:, :H]

    return pl.pallas_call(
        body,
        grid=(10,),
        in_specs=[
            pl.BlockSpec((NP // 10, F), lambda i: (i, 0)),
            pl.BlockSpec((NP // 10, F), lambda i: (i + 10, 0)),
        ],
        out_specs=pl.BlockSpec((NP // 10, H), lambda i: (i, 0)),
        out_shape=jax.ShapeDtypeStruct((NP, H), _f32),
    )(partials, partials)


def kernel(node_feats, edge_feats, edge_index, W_i, msg_W1, msg_b1, msg_W2,
           msg_b2, attn_W1, attn_b1, attn_W2, attn_b2, gru_Wih, gru_bih,
           gru_Whh, gru_bhh):
    src3 = edge_index[0].reshape(NW, NITX, CHX)
    dst3 = edge_index[1].reshape(NW, NITX, CHX)

    wab = jnp.concatenate([W_i[:128], W_i[128:256]], axis=1)
    wc = W_i[256:]
    rm = jnp.repeat(jnp.eye(H, dtype=_f32), H, axis=1)
    zrow = jnp.zeros((H, H), _f32)
    w1cat = jnp.concatenate([
        jnp.concatenate([msg_W1, attn_W1], axis=1),
        jnp.zeros((H, 2 * H), _f32)], axis=0)
    b1cat = jnp.concatenate([msg_b1, attn_b1]).reshape(1, 2 * H)
    w2blk = jnp.concatenate([
        jnp.concatenate([msg_W2, jnp.zeros((H, H * H), _f32)], axis=1),
        jnp.concatenate([jnp.zeros((H, H * H), _f32), attn_W2], axis=1),
    ], axis=0)
    b2cat = jnp.concatenate([msg_b2, attn_b2]).reshape(1, F)
    rm2 = jnp.concatenate([
        jnp.concatenate([rm, rm], axis=1),
        jnp.zeros((H, F), _f32)], axis=0)
    rt = rm.T
    wir, wiz, win = (gru_Wih[:, :H], gru_Wih[:, H:2 * H], gru_Wih[:, 2 * H:])
    whr = jnp.concatenate([gru_Whh[:, :H], zrow], axis=0)
    whz = jnp.concatenate([gru_Whh[:, H:2 * H], zrow], axis=0)
    whn = jnp.concatenate([gru_Whh[:, 2 * H:], zrow], axis=0)
    gbias = jnp.concatenate([
        gru_bih[:H] + gru_bhh[:H],
        gru_bih[H:2 * H] + gru_bhh[H:2 * H],
        gru_bih[2 * H:],
        gru_bhh[2 * H:]]).reshape(1, 4 * H)
    cw = (w1cat, b1cat, w2blk, b2cat, rm2, rt, wir, wiz, win, whr, whz, whn,
          gbias)

    t128 = _node_proj(node_feats, wab)
    g_s = _sc_gather(t128, src3, F)
    g_d = _sc_gather(t128, dst3, F)
    ef16 = _init_ef(g_s, g_d, edge_feats, wc)
    ief16 = ef16

    for step in range(3):
        payload = _pass1(ef16, w1cat, b1cat, w2blk, b2cat, rm2)
        sm = _combine_partials(_sc_scatter(payload, dst3, F), F)
        g = _sc_gather(sm, src3, F)
        newef = _pass2_gru(g, ef16, ief16, cw, out_3d=(step == 2))
        if step < 2:
            ef16 = newef

    return _readout(_sc_scatter(newef, dst3, F))[:N]


# speedup vs baseline: 58.0371x; 1.0379x over previous
import jax
import jax.numpy as jnp
from jax import lax
from jax.experimental import pallas as pl
from jax.experimental.pallas import tpu as pltpu
from jax.experimental.pallas import tpu_sc as plsc

N = 10000
NP = 10240
E = 160000
H = 8
F = 2 * H * H
NC = 2
NS = 16
NW = NC * NS
EPW = E // NW
CHX = 100
NITX = EPW // CHX
NB = 2
CHB = 200
CH2 = 100
NIT2 = EPW // CHB
NPT = NP // NS

_f32 = jnp.float32
_MESH = dict(core_axis_name="c", subcore_axis_name="s")


def _sc_scatter(payload3, dstidx3, f):
    def body(p_hbm, idx_hbm, z_hbm, out_hbm, idx_sl, b0, b1,
             l0, l1, s0, s1, acc_sh):
        bufs = (b0, b1)
        lsems = (l0, l1)
        ssems = (s0, s1)
        c = lax.axis_index("c")
        s = lax.axis_index("s")
        wid = c * NS + s
        chunk0 = wid * NITX
        nsl = pl.ds(s * NPT, NPT)
        pltpu.sync_copy(z_hbm.at[nsl], acc_sh.at[nsl])
        pltpu.sync_copy(idx_hbm.at[wid], idx_sl)
        plsc.subcore_barrier()

        def load(i, b):
            pltpu.async_copy(p_hbm.at[chunk0 + i], bufs[b], lsems[b])

        for b in range(NB - 1):
            load(b, b)

        def outer(g, carry):
            i0 = g * NB
            for b in range(NB):
                i = i0 + b
                pltpu.make_async_copy(p_hbm.at[chunk0 + i], bufs[b],
                                      lsems[b]).wait()
                pltpu.async_copy(bufs[b], acc_sh.at[idx_sl.at[i]], ssems[b],
                                 add=True)
                nxt = i + NB - 1
                bn = (b + NB - 1) % NB

                @pl.when(nxt < NITX)
                def _():
                    @pl.when(i >= 1)
                    def _():
                        pltpu.make_async_copy(
                            bufs[bn], acc_sh.at[idx_sl.at[i - 1]],
                            ssems[bn]).wait()
                    load(nxt, bn)
            return carry

        lax.fori_loop(0, NITX // NB, outer, 0)
        for b in range(NB):
            i = NITX - NB + b
            pltpu.make_async_copy(bufs[b], acc_sh.at[idx_sl.at[i]],
                                  ssems[b]).wait()
        plsc.subcore_barrier()
        pltpu.sync_copy(acc_sh.at[nsl], out_hbm.at[pl.ds(c * NP + s * NPT, NPT)])

    zeros = jnp.zeros((NP, f), _f32)
    return pl.kernel(
        body,
        out_type=jax.ShapeDtypeStruct((2 * NP, f), _f32),
        mesh=plsc.VectorSubcoreMesh(**_MESH),
        scratch_types=(
            [pltpu.VMEM((NITX, CHX), jnp.int32)]
            + [pltpu.VMEM((CHX, f), _f32) for _ in range(NB)]
            + [pltpu.SemaphoreType.DMA] * (2 * NB)
            + [pltpu.VMEM_SHARED((NP, f), _f32)]
        ),
    )(payload3, dstidx3, zeros)


def _sc_gather(table, srcidx4, f):
    def body(t_hbm, idx_hbm, out_hbm, idx_sl, b0, b1, g0, g1, t0, t1):
        bufs = (b0, b1)
        gsems = (g0, g1)
        stsems = (t0, t1)
        c = lax.axis_index("c")
        s = lax.axis_index("s")
        wid = c * NS + s
        ebase = wid * EPW
        pltpu.sync_copy(idx_hbm.at[wid], idx_sl)

        def gather(i, bi):
            pltpu.async_copy(t_hbm.at[idx_sl.at[i, 0]],
                             bufs[bi].at[pl.ds(0, CH2)], gsems[bi])
            pltpu.async_copy(t_hbm.at[idx_sl.at[i, 1]],
                             bufs[bi].at[pl.ds(CH2, CH2)], gsems[bi])

        def wait_gather(i, bi):
            pltpu.make_async_copy(t_hbm.at[idx_sl.at[i, 0]],
                                  bufs[bi].at[pl.ds(0, CH2)], gsems[bi]).wait()
            pltpu.make_async_copy(t_hbm.at[idx_sl.at[i, 1]],
                                  bufs[bi].at[pl.ds(CH2, CH2)], gsems[bi]).wait()

        def store(i, bi):
            pltpu.async_copy(bufs[bi], out_hbm.at[pl.ds(ebase + i * CHB, CHB)],
                             stsems[bi])

        def wait_store(i, bi):
            pltpu.make_async_copy(bufs[bi],
                                  out_hbm.at[pl.ds(ebase + i * CHB, CHB)],
                                  stsems[bi]).wait()

        gather(0, 0)

        def outer(g, carry):
            i = 2 * g
            wait_gather(i, 0)

            @pl.when(g >= 1)
            def _():
                wait_store(i - 1, 1)

            gather(i + 1, 1)
            store(i, 0)
            wait_gather(i + 1, 1)

            @pl.when(i + 2 < NIT2)
            def _():
                wait_store(i, 0)
                gather(i + 2, 0)

            store(i + 1, 1)
            return carry

        lax.fori_loop(0, NIT2 // 2, outer, 0)
        wait_gather(NIT2 - 1, 0)
        store(NIT2 - 1, 0)
        wait_store(NIT2 - 2, 1)
        wait_store(NIT2 - 1, 0)

    return pl.kernel(
        body,
        out_type=jax.ShapeDtypeStruct((E, f), _f32),
        mesh=plsc.VectorSubcoreMesh(**_MESH),
        scratch_types=(
            [pltpu.VMEM((NIT2, 2, CH2), jnp.int32)]
            + [pltpu.VMEM((CHB, f), _f32) for _ in range(2)]
            + [pltpu.SemaphoreType.DMA] * 4
        ),
    )(table, srcidx4)


BE = 2000
GE = E // BE
CPB = BE // CHX


def _full(shape):
    nd = len(shape)
    return pl.BlockSpec(shape, lambda i: (0,) * nd)


def _blk(shape):
    return pl.BlockSpec(shape, lambda i: (i,) + (0,) * (len(shape) - 1))


def _node_proj(node_feats, wab):
    def body(nf_ref, w_ref, out_ref):
        ab = jnp.dot(nf_ref[...], w_ref[...], preferred_element_type=_f32)
        out_ref[...] = jnp.concatenate(
            [ab, jnp.zeros((ab.shape[0], F - 2 * H), _f32)], axis=1)

    return pl.pallas_call(
        body,
        grid=(10,),
        in_specs=[_blk((N // 10, 128)), _full((128, 2 * H))],
        out_specs=_blk((N // 10, F)),
        out_shape=jax.ShapeDtypeStruct((NP, F), _f32),
    )(node_feats, wab)


def _init_ef(g_s, g_d, edge_feats, wc):
    def body(s_ref, d_ref, ef_ref, w_ref, out_ref):
        x = (s_ref[...][:, :H] + d_ref[...][:, H:2 * H]
             + jnp.dot(ef_ref[...], w_ref[...], preferred_element_type=_f32))
        x = jnp.maximum(x, 0.0)
        out_ref[...] = jnp.concatenate([x, jnp.zeros_like(x)], axis=1)

    return pl.pallas_call(
        body,
        grid=(GE,),
        in_specs=[_blk((BE, F)), _blk((BE, F)), _blk((BE, 16)),
                  _full((16, H))],
        out_specs=_blk((BE, 2 * H)),
        out_shape=jax.ShapeDtypeStruct((E, 2 * H), _f32),
    )(g_s, g_d, edge_feats, wc)


def _edge_mats(ef16, w1cat, b1cat, w2blk, b2cat, rm2):
    t = jnp.dot(ef16, w1cat, preferred_element_type=_f32) + b1cat
    u = jnp.maximum(t, 0.0)
    wma = jnp.dot(u, w2blk, preferred_element_type=_f32) + b2cat
    ef_r2 = jnp.dot(ef16, rm2, preferred_element_type=_f32)
    return wma, wma * ef_r2


def _pass1(ef16, w1cat, b1cat, w2blk, b2cat, rm2):
    def body(ef_ref, w1_ref, b1_ref, w2_ref, b2_ref, rm2_ref, out_ref):
        _, e_all = _edge_mats(ef_ref[...], w1_ref[...], b1_ref[...],
                              w2_ref[...], b2_ref[...], rm2_ref[...])
        exp_e2 = jnp.exp(e_all[:, H * H:])
        h1 = exp_e2 * e_all[:, :H * H]
        out_ref[...] = jnp.concatenate([exp_e2, h1],
                                       axis=1).reshape(CPB, CHX, F)

    return pl.pallas_call(
        body,
        grid=(GE,),
        in_specs=[_blk((BE, 2 * H)), _full((2 * H, 2 * H)), _full((1, 2 * H)),
                  _full((2 * H, F)), _full((1, F)), _full((2 * H, F))],
        out_specs=_blk((CPB, CHX, F)),
        out_shape=jax.ShapeDtypeStruct((NW * NITX, CHX, F), _f32),
    )(ef16, w1cat, b1cat, w2blk, b2cat, rm2)


def _combine_partials(partials, f):
    def body(a_ref, b_ref, out_ref):
        out_ref[...] = a_ref[...] + b_ref[...]

    return pl.pallas_call(
        body,
        grid=(10,),
        in_specs=[
            pl.BlockSpec((NP // 10, f), lambda i: (i, 0)),
            pl.BlockSpec((NP // 10, f), lambda i: (i + 10, 0)),
        ],
        out_specs=pl.BlockSpec((NP // 10, f), lambda i: (i, 0)),
        out_shape=jax.ShapeDtypeStruct((NP, f), _f32),
    )(partials, partials)


def _pass2_gru(g, ef16, ief16, cw, out_3d=False):
    def body(g_ref, ef_ref, ief_ref, w1_ref, b1_ref, w2_ref, b2_ref, rm2_ref,
             rt_ref, wir_ref, wiz_ref, win_ref, whr_ref, whz_ref, whn_ref,
             gb_ref, out_ref):
        ef16v = ef_ref[...]
        wma, e_all = _edge_mats(ef16v, w1_ref[...], b1_ref[...], w2_ref[...],
                                b2_ref[...], rm2_ref[...])
        ie_all = wma * jnp.dot(ief_ref[...], rm2_ref[...],
                               preferred_element_type=_f32)
        exp_e2 = jnp.exp(e_all[:, H * H:])
        h1 = exp_e2 * e_all[:, :H * H]
        exp_ie2 = jnp.exp(ie_all[:, H * H:])
        ih1 = exp_ie2 * ie_all[:, :H * H]
        gathered = g_ref[...]
        sg = gathered[:, :H * H]
        mg = gathered[:, H * H:]
        h2 = (mg - h1 + ih1) / (sg - exp_e2 + exp_ie2)
        conv = jnp.maximum(jnp.dot(h2, rt_ref[...],
                                   preferred_element_type=_f32), 0.0)
        gb = gb_ref[...]
        r = jax.nn.sigmoid(
            jnp.dot(conv, wir_ref[...], preferred_element_type=_f32)
            + jnp.dot(ef16v, whr_ref[...], preferred_element_type=_f32)
            + gb[:, :H])
        z = jax.nn.sigmoid(
            jnp.dot(conv, wiz_ref[...], preferred_element_type=_f32)
            + jnp.dot(ef16v, whz_ref[...], preferred_element_type=_f32)
            + gb[:, H:2 * H])
        n = jnp.tanh(
            jnp.dot(conv, win_ref[...], preferred_element_type=_f32)
            + gb[:, 2 * H:3 * H]
            + r * (jnp.dot(ef16v, whn_ref[...], preferred_element_type=_f32)
                   + gb[:, 3 * H:]))
        newef = (1.0 - z) * n + z * ef16v[:, :H]
        if out_3d:
            out_ref[...] = jnp.concatenate(
                [newef, jnp.zeros((BE, F - H), _f32)],
                axis=1).reshape(CPB, CHX, F)
        else:
            out_ref[...] = jnp.concatenate([newef, jnp.zeros_like(newef)],
                                           axis=1)

    if out_3d:
        out_spec = _blk((CPB, CHX, F))
        out_shape = jax.ShapeDtypeStruct((NW * NITX, CHX, F), _f32)
    else:
        out_spec = _blk((BE, 2 * H))
        out_shape = jax.ShapeDtypeStruct((E, 2 * H), _f32)
    (w1cat, b1cat, w2blk, b2cat, rm2, rt, wir, wiz, win, whr, whz, whn,
     gbias) = cw
    return pl.pallas_call(
        body,
        grid=(GE,),
        in_specs=[_blk((BE, F)), _blk((BE, 2 * H)), _blk((BE, 2 * H)),
                  _full((2 * H, 2 * H)), _full((1, 2 * H)), _full((2 * H, F)),
                  _full((1, F)), _full((2 * H, F)), _full((H * H, H)),
                  _full((H, H)), _full((H, H)), _full((H, H)),
                  _full((2 * H, H)), _full((2 * H, H)), _full((2 * H, H)),
                  _full((1, 4 * H))],
        out_specs=out_spec,
        out_shape=out_shape,
    )(g, ef16, ief16, w1cat, b1cat, w2blk, b2cat, rm2, rt, wir, wiz, win,
      whr, whz, whn, gbias)


def _readout(partials):
    def body(a_ref, b_ref, out_ref):
        out_ref[...] = (a_ref[...] + b_ref[...])[:, :H]

    return pl.pallas_call(
        body,
        grid=(10,),
        in_specs=[
            pl.BlockSpec((NP // 10, F), lambda i: (i, 0)),
            pl.BlockSpec((NP // 10, F), lambda i: (i + 10, 0)),
        ],
        out_specs=pl.BlockSpec((NP // 10, H), lambda i: (i, 0)),
        out_shape=jax.ShapeDtypeStruct((NP, H), _f32),
    )(partials, partials)


def kernel(node_feats, edge_feats, edge_index, W_i, msg_W1, msg_b1, msg_W2,
           msg_b2, attn_W1, attn_b1, attn_W2, attn_b2, gru_Wih, gru_bih,
           gru_Whh, gru_bhh):
    src4 = edge_index[0].reshape(NW, NIT2, 2, CH2)
    dst4 = edge_index[1].reshape(NW, NIT2, 2, CH2)
    dst3 = edge_index[1].reshape(NW, NITX, CHX)

    wab = jnp.concatenate([W_i[:128], W_i[128:256]], axis=1)
    wc = W_i[256:]
    rm = jnp.repeat(jnp.eye(H, dtype=_f32), H, axis=1)
    zrow = jnp.zeros((H, H), _f32)
    w1cat = jnp.concatenate([
        jnp.concatenate([msg_W1, attn_W1], axis=1),
        jnp.zeros((H, 2 * H), _f32)], axis=0)
    b1cat = jnp.concatenate([msg_b1, attn_b1]).reshape(1, 2 * H)
    w2blk = jnp.concatenate([
        jnp.concatenate([msg_W2, jnp.zeros((H, H * H), _f32)], axis=1),
        jnp.concatenate([jnp.zeros((H, H * H), _f32), attn_W2], axis=1),
    ], axis=0)
    b2cat = jnp.concatenate([msg_b2, attn_b2]).reshape(1, F)
    rm2 = jnp.concatenate([
        jnp.concatenate([rm, rm], axis=1),
        jnp.zeros((H, F), _f32)], axis=0)
    rt = rm.T
    wir, wiz, win = (gru_Wih[:, :H], gru_Wih[:, H:2 * H], gru_Wih[:, 2 * H:])
    whr = jnp.concatenate([gru_Whh[:, :H], zrow], axis=0)
    whz = jnp.concatenate([gru_Whh[:, H:2 * H], zrow], axis=0)
    whn = jnp.concatenate([gru_Whh[:, 2 * H:], zrow], axis=0)
    gbias = jnp.concatenate([
        gru_bih[:H] + gru_bhh[:H],
        gru_bih[H:2 * H] + gru_bhh[H:2 * H],
        gru_bih[2 * H:],
        gru_bhh[2 * H:]]).reshape(1, 4 * H)
    cw = (w1cat, b1cat, w2blk, b2cat, rm2, rt, wir, wiz, win, whr, whz, whn,
          gbias)

    t128 = _node_proj(node_feats, wab)
    g_s = _sc_gather(t128, src4, F)
    g_d = _sc_gather(t128, dst4, F)
    ef16 = _init_ef(g_s, g_d, edge_feats, wc)
    ief16 = ef16

    for step in range(3):
        payload = _pass1(ef16, w1cat, b1cat, w2blk, b2cat, rm2)
        sm = _combine_partials(_sc_scatter(payload, dst3, F), F)
        g = _sc_gather(sm, src4, F)
        newef = _pass2_gru(g, ef16, ief16, cw, out_3d=(step == 2))
        if step < 2:
            ef16 = newef

    return _readout(_sc_scatter(newef, dst3, F))[:N]
